# Initial kernel scaffold; baseline (speedup 1.0000x reference)
import jax
import jax.numpy as jnp
from jax.experimental import pallas as pl


def kernel(x, h_initial, target_params, predictor_params):
    raise NotImplementedError("write your pallas kernel here")


# speedup vs baseline: 22.9937x; 22.9937x over previous
import jax
import jax.numpy as jnp
import numpy as np
from jax.experimental import pallas as pl
from jax.experimental.pallas import tpu as pltpu

NP = 55
ND = 3
NH = 32
NL = 3
NB = 256
NPAIR = NP * NP
CRL = 15.0 / NL


def _selection_mats():
    ii = np.arange(NPAIR) // NP
    jj = np.arange(NPAIR) % NP
    n = np.arange(NP)
    s_row = (ii[None, :] == n[:, None]).astype(np.float32)
    s_col = (jj[None, :] == n[:, None]).astype(np.float32)
    s_agg_t = ((ii[:, None] == n[None, :]) & (ii != jj)[:, None]
               ).astype(np.float32)
    return jnp.asarray(s_row), jnp.asarray(s_col), jnp.asarray(s_agg_t)


def _flatten_net(p):
    out = [p["embedding"]["W"],
           p["embedding"]["b"].reshape(NH, 1)]
    for lp in p["layers"]:
        out.extend([
            lp["edge_mlp0"]["W"],
            lp["edge_mlp0"]["b"].reshape(NH, 1),
            lp["edge_mlp1"]["W"],
            lp["edge_mlp1"]["b"].reshape(NH, 1),
            lp["att_mlp"]["W"],
            lp["att_mlp"]["b"].reshape(1, 1),
            lp["coord_mlp0"]["W"],
            lp["coord_mlp0"]["b"].reshape(NH, 1),
            lp["coord_mlp1"]["W"],
            lp["node_mlp0"]["W"],
            lp["node_mlp0"]["b"].reshape(NH, 1),
            lp["node_mlp1"]["W"],
            lp["node_mlp1"]["b"].reshape(NH, 1),
        ])
    return out

_PER_LAYER = 13
_PER_NET = 2 + NL * _PER_LAYER


def _dot(a, b):
    return jax.lax.dot(a, b, preferred_element_type=jnp.float32)


def _run_net(prm, hi, x0, ea, s_row, s_col, s_agg_t):
    emb_w = prm[0][...]
    emb_b = prm[1][...]
    h = emb_w[:, 0:1] * hi + emb_b
    xc = x0
    for l in range(NL):
        o = 2 + l * _PER_LAYER
        e0w = prm[o + 0][...]
        e0b = prm[o + 1][...]
        e1w = prm[o + 2][...]
        e1b = prm[o + 3][...]
        attw = prm[o + 4][...]
        attb = prm[o + 5][...]
        c0w = prm[o + 6][...]
        c0b = prm[o + 7][...]
        c1w = prm[o + 8][...]
        n0w = prm[o + 9][...]
        n0b = prm[o + 10][...]
        n1w = prm[o + 11][...]
        n1b = prm[o + 12][...]

        d = _dot(xc, s_row) - _dot(xc, s_col)
        radial = jnp.sum(d * d, axis=0, keepdims=True)
        u = d / (jnp.sqrt(radial) + 1.0)

        pre = (_dot(_dot(e0w[:, :NH], h), s_row)
               + _dot(_dot(e0w[:, NH:2 * NH], h), s_col)
               + e0w[:, 2 * NH:2 * NH + 1] * radial
               + e0w[:, 2 * NH + 1:] * ea
               + e0b)
        e1 = jax.nn.silu(pre)
        e2 = jax.nn.silu(_dot(e1w, e1) + e1b)
        att = jax.nn.sigmoid(_dot(attw, e2) + attb)
        ef = e2 * att
        m1 = jax.nn.silu(_dot(c0w, ef) + c0b)
        m2 = jnp.tanh(_dot(c1w, m1))
        xc = xc + _dot(u * m2, s_agg_t) * CRL

        if l < NL - 1:
            agg_h = _dot(ef, s_agg_t)
            nh = jax.nn.silu(_dot(n0w[:, :NH], h)
                             + _dot(n0w[:, NH:], agg_h) + n0b)
            h = h + _dot(n1w, nh) + n1b
    return xc


def _body(x_ref, hi_ref, sr_ref, sc_ref, sa_ref, *refs):
    prm = refs[:2 * _PER_NET]
    out_ref = refs[2 * _PER_NET]
    x0 = x_ref[0]
    hi = hi_ref[...]
    s_row = sr_ref[...]
    s_col = sc_ref[...]
    s_agg_t = sa_ref[...]
    d0 = _dot(x0, s_row) - _dot(x0, s_col)
    ea = jnp.sum(d0 * d0, axis=0, keepdims=True)
    x_t = _run_net(prm[:_PER_NET], hi, x0, ea, s_row, s_col, s_agg_t)
    x_p = _run_net(prm[_PER_NET:], hi, x0, ea, s_row, s_col, s_agg_t)
    diff = x_p - x_t
    out_ref[0] = jnp.sum(diff * diff)[None, None]


def kernel(x, h_initial, target_params, predictor_params):
    x3 = x.reshape(NB, NP, ND).transpose(0, 2, 1)
    hi = h_initial.reshape(1, NP)
    s_row, s_col, s_agg_t = _selection_mats()
    prm = _flatten_net(target_params) + _flatten_net(predictor_params)

    const = lambda shape: pl.BlockSpec(shape, lambda b: (0,) * len(shape))
    in_specs = [
        pl.BlockSpec((1, ND, NP), lambda b: (b, 0, 0)),
        const((1, NP)),
        const((NP, NPAIR)),
        const((NP, NPAIR)),
        const((NPAIR, NP)),
    ] + [const(p.shape) for p in prm]

    out = pl.pallas_call(
        _body,
        grid=(NB,),
        in_specs=in_specs,
        out_specs=pl.BlockSpec((1, 1, 1), lambda b: (b, 0, 0)),
        out_shape=jax.ShapeDtypeStruct((NB, 1, 1), jnp.float32),
        compiler_params=pltpu.CompilerParams(
            dimension_semantics=("arbitrary",),
        ),
    )(x3, hi, s_row, s_col, s_agg_t, *prm)
    return out[:, 0, 0]


# speedup vs baseline: 25.4664x; 1.1075x over previous
import jax
import jax.numpy as jnp
import numpy as np
from jax.experimental import pallas as pl
from jax.experimental.pallas import tpu as pltpu

NP = 55
ND = 3
NH = 32
NL = 3
NB = 256
NPAIR = NP * NP
CRL = 15.0 / NL


def _selection_mats():
    ii = np.arange(NPAIR) // NP
    jj = np.arange(NPAIR) % NP
    n = np.arange(NP)
    s_row = (ii[None, :] == n[:, None]).astype(np.float32)
    s_col = (jj[None, :] == n[:, None]).astype(np.float32)
    s_bc = np.vstack([s_row, s_col])
    s_d = s_row - s_col
    s_agg_t = ((ii[:, None] == n[None, :]) & (ii != jj)[:, None]
               ).astype(np.float32)
    return jnp.asarray(s_bc), jnp.asarray(s_d), jnp.asarray(s_agg_t)


def _flatten_net(p):
    out = [p["embedding"]["W"],
           p["embedding"]["b"].reshape(NH, 1)]
    for lp in p["layers"]:
        out.extend([
            lp["edge_mlp0"]["W"],
            lp["edge_mlp0"]["b"].reshape(NH, 1),
            lp["edge_mlp1"]["W"],
            lp["edge_mlp1"]["b"].reshape(NH, 1),
            lp["att_mlp"]["W"],
            lp["att_mlp"]["b"].reshape(1, 1),
            lp["coord_mlp0"]["W"],
            lp["coord_mlp0"]["b"].reshape(NH, 1),
            lp["coord_mlp1"]["W"],
            lp["node_mlp0"]["W"],
            lp["node_mlp0"]["b"].reshape(NH, 1),
            lp["node_mlp1"]["W"],
            lp["node_mlp1"]["b"].reshape(NH, 1),
        ])
    return out

_PER_LAYER = 13
_PER_NET = 2 + NL * _PER_LAYER


def _dot(a, b):
    return jax.lax.dot(a, b, preferred_element_type=jnp.float32)


def _body(x_ref, hi_ref, sbc_ref, sd_ref, sa_ref, *refs):
    prm = refs[:2 * _PER_NET]
    out_ref = refs[2 * _PER_NET]
    x0 = x_ref[0]
    hi = hi_ref[...]
    s_bc = sbc_ref[...]
    s_d = sd_ref[...]
    s_agg_t = sa_ref[...]

    d0 = _dot(x0, s_d)
    ea = jnp.sum(d0 * d0, axis=0, keepdims=True)
    u0 = d0 / (jnp.sqrt(ea) + 1.0)

    nets = []
    for pn in (prm[:_PER_NET], prm[_PER_NET:]):
        emb_w = pn[0][...]
        emb_b = pn[1][...]
        nets.append({"p": pn, "h": emb_w[:, 0:1] * hi + emb_b, "x": x0})

    for l in range(NL):
        for st in nets:
            p = st["p"]
            o = 2 + l * _PER_LAYER
            e0w = p[o + 0][...]
            e0b = p[o + 1][...]
            e1w = p[o + 2][...]
            e1b = p[o + 3][...]
            attw = p[o + 4][...]
            attb = p[o + 5][...]
            c0w = p[o + 6][...]
            c0b = p[o + 7][...]
            c1w = p[o + 8][...]
            h = st["h"]
            xc = st["x"]

            if l == 0:
                radial, u = ea, u0
            else:
                d = _dot(xc, s_d)
                radial = jnp.sum(d * d, axis=0, keepdims=True)
                u = d / (jnp.sqrt(radial) + 1.0)

            acat = jnp.concatenate(
                [_dot(e0w[:, :NH], h), _dot(e0w[:, NH:2 * NH], h)], axis=1)
            pre = (_dot(acat, s_bc)
                   + e0w[:, 2 * NH:2 * NH + 1] * radial
                   + e0w[:, 2 * NH + 1:] * ea
                   + e0b)
            e1 = jax.nn.silu(pre)
            e2 = jax.nn.silu(_dot(e1w, e1) + e1b)
            att = jax.nn.sigmoid(_dot(attw, e2) + attb)
            ef = e2 * att
            m1 = jax.nn.silu(_dot(c0w, ef) + c0b)
            m2 = jnp.tanh(_dot(c1w, m1))

            if l < NL - 1:
                y = _dot(jnp.concatenate([ef, u * m2], axis=0), s_agg_t)
                st["x"] = xc + y[NH:NH + ND] * CRL
                agg_h = y[:NH]
                n0w = p[o + 9][...]
                n0b = p[o + 10][...]
                n1w = p[o + 11][...]
                n1b = p[o + 12][...]
                nh = jax.nn.silu(_dot(n0w[:, :NH], h)
                                 + _dot(n0w[:, NH:], agg_h) + n0b)
                st["h"] = h + _dot(n1w, nh) + n1b
            else:
                st["x"] = xc + _dot(u * m2, s_agg_t) * CRL

    diff = nets[0]["x"] - nets[1]["x"]
    out_ref[0] = jnp.sum(diff * diff)[None, None]


def kernel(x, h_initial, target_params, predictor_params):
    x3 = x.reshape(NB, NP, ND).transpose(0, 2, 1)
    hi = h_initial.reshape(1, NP)
    s_bc, s_d, s_agg_t = _selection_mats()
    prm = _flatten_net(target_params) + _flatten_net(predictor_params)

    const = lambda shape: pl.BlockSpec(shape, lambda b: (0,) * len(shape))
    in_specs = [
        pl.BlockSpec((1, ND, NP), lambda b: (b, 0, 0)),
        const((1, NP)),
        const((2 * NP, NPAIR)),
        const((NP, NPAIR)),
        const((NPAIR, NP)),
    ] + [const(p.shape) for p in prm]

    out = pl.pallas_call(
        _body,
        grid=(NB,),
        in_specs=in_specs,
        out_specs=pl.BlockSpec((1, 1, 1), lambda b: (b, 0, 0)),
        out_shape=jax.ShapeDtypeStruct((NB, 1, 1), jnp.float32),
        compiler_params=pltpu.CompilerParams(
            dimension_semantics=("parallel",),
        ),
    )(x3, hi, s_bc, s_d, s_agg_t, *prm)
    return out[:, 0, 0]


# speedup vs baseline: 27.0553x; 1.0624x over previous
import jax
import jax.numpy as jnp
import numpy as np
from jax.experimental import pallas as pl
from jax.experimental.pallas import tpu as pltpu

NP = 55
ND = 3
NH = 32
NL = 3
NB = 256
NPAIR = NP * NP
CRL = 15.0 / NL


def _selection_mats():
    ii = np.arange(NPAIR) // NP
    jj = np.arange(NPAIR) % NP
    n = np.arange(NP)
    s_row = (ii[None, :] == n[:, None]).astype(np.float32)
    s_col = (jj[None, :] == n[:, None]).astype(np.float32)
    s_bc = np.vstack([s_row, s_col])
    s_d = s_row - s_col
    s_agg_t = ((ii[:, None] == n[None, :]) & (ii != jj)[:, None]
               ).astype(np.float32)
    return jnp.asarray(s_bc), jnp.asarray(s_d), jnp.asarray(s_agg_t)


def _flatten_net(p):
    out = [p["embedding"]["W"],
           p["embedding"]["b"].reshape(NH, 1)]
    for lp in p["layers"]:
        out.extend([
            lp["edge_mlp0"]["W"],
            lp["edge_mlp0"]["b"].reshape(NH, 1),
            lp["edge_mlp1"]["W"],
            lp["edge_mlp1"]["b"].reshape(NH, 1),
            lp["att_mlp"]["W"],
            lp["att_mlp"]["b"].reshape(1, 1),
            lp["coord_mlp0"]["W"],
            lp["coord_mlp0"]["b"].reshape(NH, 1),
            lp["coord_mlp1"]["W"],
            lp["node_mlp0"]["W"],
            lp["node_mlp0"]["b"].reshape(NH, 1),
            lp["node_mlp1"]["W"],
            lp["node_mlp1"]["b"].reshape(NH, 1),
        ])
    return out

_PER_LAYER = 13
_PER_NET = 2 + NL * _PER_LAYER


def _dot(a, b):
    return jax.lax.dot(a, b, preferred_element_type=jnp.float32)


GS = 2


def _body(x_ref, hi_ref, sbc_ref, sd_ref, sa_ref, *refs):
    prm = refs[:2 * _PER_NET]
    out_ref = refs[2 * _PER_NET]
    hi = hi_ref[...]
    s_bc = sbc_ref[...]
    s_d = sd_ref[...]
    s_agg_t = sa_ref[...]

    nets = []
    for g in range(GS):
        x0 = x_ref[g]
        d0 = _dot(x0, s_d)
        ea = jnp.sum(d0 * d0, axis=0, keepdims=True)
        u0 = d0 / (jnp.sqrt(ea) + 1.0)
        for pn in (prm[:_PER_NET], prm[_PER_NET:]):
            emb_w = pn[0][...]
            emb_b = pn[1][...]
            nets.append({"p": pn, "h": emb_w[:, 0:1] * hi + emb_b,
                         "x": x0, "ea": ea, "u0": u0})

    for l in range(NL):
        for st in nets:
            p = st["p"]
            o = 2 + l * _PER_LAYER
            e0w = p[o + 0][...]
            e0b = p[o + 1][...]
            e1w = p[o + 2][...]
            e1b = p[o + 3][...]
            attw = p[o + 4][...]
            attb = p[o + 5][...]
            c0w = p[o + 6][...]
            c0b = p[o + 7][...]
            c1w = p[o + 8][...]
            h = st["h"]
            xc = st["x"]
            ea = st["ea"]

            if l == 0:
                radial, u = ea, st["u0"]
            else:
                d = _dot(xc, s_d)
                radial = jnp.sum(d * d, axis=0, keepdims=True)
                u = d / (jnp.sqrt(radial) + 1.0)

            acat = jnp.concatenate(
                [_dot(e0w[:, :NH], h), _dot(e0w[:, NH:2 * NH], h)], axis=1)
            pre = (_dot(acat, s_bc)
                   + e0w[:, 2 * NH:2 * NH + 1] * radial
                   + e0w[:, 2 * NH + 1:] * ea
                   + e0b)
            e1 = jax.nn.silu(pre)
            e2 = jax.nn.silu(_dot(e1w, e1) + e1b)
            att = jax.nn.sigmoid(_dot(attw, e2) + attb)
            ef = e2 * att
            m1 = jax.nn.silu(_dot(c0w, ef) + c0b)
            m2 = jnp.tanh(_dot(c1w, m1))

            if l < NL - 1:
                y = _dot(jnp.concatenate([ef, u * m2], axis=0), s_agg_t)
                st["x"] = xc + y[NH:NH + ND] * CRL
                agg_h = y[:NH]
                n0w = p[o + 9][...]
                n0b = p[o + 10][...]
                n1w = p[o + 11][...]
                n1b = p[o + 12][...]
                nh = jax.nn.silu(_dot(n0w[:, :NH], h)
                                 + _dot(n0w[:, NH:], agg_h) + n0b)
                st["h"] = h + _dot(n1w, nh) + n1b
            else:
                st["x"] = xc + _dot(u * m2, s_agg_t) * CRL

    for g in range(GS):
        diff = nets[2 * g]["x"] - nets[2 * g + 1]["x"]
        out_ref[g] = jnp.sum(diff * diff)[None, None]


def kernel(x, h_initial, target_params, predictor_params):
    x3 = x.reshape(NB, NP, ND).transpose(0, 2, 1)
    hi = h_initial.reshape(1, NP)
    s_bc, s_d, s_agg_t = _selection_mats()
    prm = _flatten_net(target_params) + _flatten_net(predictor_params)

    const = lambda shape: pl.BlockSpec(shape, lambda b: (0,) * len(shape))
    in_specs = [
        pl.BlockSpec((GS, ND, NP), lambda b: (b, 0, 0)),
        const((1, NP)),
        const((2 * NP, NPAIR)),
        const((NP, NPAIR)),
        const((NPAIR, NP)),
    ] + [const(p.shape) for p in prm]

    out = pl.pallas_call(
        _body,
        grid=(NB // GS,),
        in_specs=in_specs,
        out_specs=pl.BlockSpec((GS, 1, 1), lambda b: (b, 0, 0)),
        out_shape=jax.ShapeDtypeStruct((NB, 1, 1), jnp.float32),
        compiler_params=pltpu.CompilerParams(
            dimension_semantics=("parallel",),
        ),
    )(x3, hi, s_bc, s_d, s_agg_t, *prm)
    return out[:, 0, 0]


# speedup vs baseline: 27.4481x; 1.0145x over previous
import jax
import jax.numpy as jnp
import numpy as np
from jax.experimental import pallas as pl
from jax.experimental.pallas import tpu as pltpu

NP = 55
ND = 3
NH = 32
NL = 3
NB = 256
NPAIR = NP * NP
CRL = 15.0 / NL


def _selection_mats():
    ii = np.arange(NPAIR) // NP
    jj = np.arange(NPAIR) % NP
    n = np.arange(NP)
    s_row = (ii[None, :] == n[:, None]).astype(np.float32)
    s_col = (jj[None, :] == n[:, None]).astype(np.float32)
    s_bc = np.vstack([s_row, s_col,
                      np.ones((1, NPAIR), np.float32)])
    s_d = s_row - s_col
    s_agg_t = ((ii[:, None] == n[None, :]) & (ii != jj)[:, None]
               ).astype(np.float32)
    return jnp.asarray(s_bc), jnp.asarray(s_d), jnp.asarray(s_agg_t)


def _flatten_net(p):
    out = [p["embedding"]["W"],
           p["embedding"]["b"].reshape(NH, 1)]
    for lp in p["layers"]:
        out.extend([
            lp["edge_mlp0"]["W"],
            lp["edge_mlp0"]["b"].reshape(NH, 1),
            lp["edge_mlp1"]["W"],
            lp["edge_mlp1"]["b"].reshape(NH, 1),
            lp["att_mlp"]["W"],
            lp["att_mlp"]["b"].reshape(1, 1),
            lp["coord_mlp0"]["W"],
            lp["coord_mlp0"]["b"].reshape(NH, 1),
            lp["coord_mlp1"]["W"],
            lp["node_mlp0"]["W"],
            lp["node_mlp0"]["b"].reshape(NH, 1),
            lp["node_mlp1"]["W"],
            lp["node_mlp1"]["b"].reshape(NH, 1),
        ])
    return out

_PER_LAYER = 13
_PER_NET = 2 + NL * _PER_LAYER


def _dot(a, b):
    return jax.lax.dot(a, b, preferred_element_type=jnp.float32)


GS = 2


def _body(x_ref, hi_ref, sbc_ref, sd_ref, sa_ref, *refs):
    prm = refs[:2 * _PER_NET]
    out_ref = refs[2 * _PER_NET]
    hi = hi_ref[...]
    s_bc = sbc_ref[...]
    s_d = sd_ref[...]
    s_agg_t = sa_ref[...]

    nets = []
    for g in range(GS):
        x0 = x_ref[g]
        d0 = _dot(x0, s_d)
        ea = jnp.sum(d0 * d0, axis=0, keepdims=True)
        u0 = d0 / (jnp.sqrt(ea) + 1.0)
        for pn in (prm[:_PER_NET], prm[_PER_NET:]):
            emb_w = pn[0][...]
            emb_b = pn[1][...]
            nets.append({"p": pn, "h": emb_w[:, 0:1] * hi + emb_b,
                         "x": x0, "ea": ea, "u0": u0})

    for l in range(NL):
        for st in nets:
            p = st["p"]
            o = 2 + l * _PER_LAYER
            e0w = p[o + 0][...]
            e0b = p[o + 1][...]
            e1w = p[o + 2][...]
            e1b = p[o + 3][...]
            attw = p[o + 4][...]
            attb = p[o + 5][...]
            c0w = p[o + 6][...]
            c0b = p[o + 7][...]
            c1w = p[o + 8][...]
            h = st["h"]
            xc = st["x"]
            ea = st["ea"]

            if l == 0:
                radial, u = ea, st["u0"]
            else:
                d = _dot(xc, s_d)
                radial = jnp.sum(d * d, axis=0, keepdims=True)
                u = d / (jnp.sqrt(radial) + 1.0)

            acat = jnp.concatenate(
                [_dot(e0w[:, :NH], h), _dot(e0w[:, NH:2 * NH], h), e0b],
                axis=1)
            re = jnp.concatenate([radial, ea], axis=0)
            pre = (_dot(acat, s_bc)
                   + _dot(e0w[:, 2 * NH:], re))
            e1 = jax.nn.silu(pre)
            e2 = jax.nn.silu(_dot(e1w, e1) + e1b)
            att = jax.nn.sigmoid(_dot(attw, e2) + attb)
            ef = e2 * att
            m1 = jax.nn.silu(_dot(c0w, ef) + c0b)
            m2 = jnp.tanh(_dot(c1w, m1))

            if l < NL - 1:
                y = _dot(jnp.concatenate([ef, u * m2], axis=0), s_agg_t)
                st["x"] = xc + y[NH:NH + ND] * CRL
                agg_h = y[:NH]
                n0w = p[o + 9][...]
                n0b = p[o + 10][...]
                n1w = p[o + 11][...]
                n1b = p[o + 12][...]
                nh = jax.nn.silu(_dot(n0w[:, :NH], h)
                                 + _dot(n0w[:, NH:], agg_h) + n0b)
                st["h"] = h + _dot(n1w, nh) + n1b
            else:
                st["x"] = xc + _dot(u * m2, s_agg_t) * CRL

    for g in range(GS):
        diff = nets[2 * g]["x"] - nets[2 * g + 1]["x"]
        out_ref[g] = jnp.sum(diff * diff)[None, None]


def kernel(x, h_initial, target_params, predictor_params):
    x3 = x.reshape(NB, NP, ND).transpose(0, 2, 1)
    hi = h_initial.reshape(1, NP)
    s_bc, s_d, s_agg_t = _selection_mats()
    prm = _flatten_net(target_params) + _flatten_net(predictor_params)

    const = lambda shape: pl.BlockSpec(shape, lambda b: (0,) * len(shape))
    in_specs = [
        pl.BlockSpec((GS, ND, NP), lambda b: (b, 0, 0)),
        const((1, NP)),
        const((2 * NP + 1, NPAIR)),
        const((NP, NPAIR)),
        const((NPAIR, NP)),
    ] + [const(p.shape) for p in prm]

    out = pl.pallas_call(
        _body,
        grid=(NB // GS,),
        in_specs=in_specs,
        out_specs=pl.BlockSpec((GS, 1, 1), lambda b: (b, 0, 0)),
        out_shape=jax.ShapeDtypeStruct((NB, 1, 1), jnp.float32),
        compiler_params=pltpu.CompilerParams(
            dimension_semantics=("parallel",),
        ),
    )(x3, hi, s_bc, s_d, s_agg_t, *prm)
    return out[:, 0, 0]


# speedup vs baseline: 28.2143x; 1.0279x over previous
import jax
import jax.numpy as jnp
import numpy as np
from jax.experimental import pallas as pl
from jax.experimental.pallas import tpu as pltpu

NP = 55
ND = 3
NH = 32
NL = 3
NB = 256
NPAIR = NP * NP
CRL = 15.0 / NL


def _selection_mats():
    ii = np.arange(NPAIR) // NP
    jj = np.arange(NPAIR) % NP
    n = np.arange(NP)
    s_row = (ii[None, :] == n[:, None]).astype(np.float32)
    s_col = (jj[None, :] == n[:, None]).astype(np.float32)
    s_bc = np.vstack([s_row, s_col,
                      np.ones((1, NPAIR), np.float32)])
    s_d = s_row - s_col
    s_agg_t = ((ii[:, None] == n[None, :]) & (ii != jj)[:, None]
               ).astype(np.float32)
    return jnp.asarray(s_bc), jnp.asarray(s_d), jnp.asarray(s_agg_t)


def _flatten_net(p):
    out = [p["embedding"]["W"],
           p["embedding"]["b"].reshape(NH, 1)]
    for lp in p["layers"]:
        out.extend([
            lp["edge_mlp0"]["W"],
            lp["edge_mlp0"]["b"].reshape(NH, 1),
            lp["edge_mlp1"]["W"],
            lp["edge_mlp1"]["b"].reshape(NH, 1),
            lp["att_mlp"]["W"],
            lp["att_mlp"]["b"].reshape(1, 1),
            lp["coord_mlp0"]["W"],
            lp["coord_mlp0"]["b"].reshape(NH, 1),
            lp["coord_mlp1"]["W"],
            lp["node_mlp0"]["W"],
            lp["node_mlp0"]["b"].reshape(NH, 1),
            lp["node_mlp1"]["W"],
            lp["node_mlp1"]["b"].reshape(NH, 1),
        ])
    return out

_PER_LAYER = 13
_PER_NET = 2 + NL * _PER_LAYER


def _dot(a, b):
    return jax.lax.dot(a, b, preferred_element_type=jnp.float32)


GS = 4


def _body(x_ref, hi_ref, sbc_ref, sd_ref, sa_ref, *refs):
    prm = refs[:2 * _PER_NET]
    out_ref = refs[2 * _PER_NET]
    hi = hi_ref[...]
    s_bc = sbc_ref[...]
    s_d = sd_ref[...]
    s_agg_t = sa_ref[...]

    nets = []
    for g in range(GS):
        x0 = x_ref[g]
        d0 = _dot(x0, s_d)
        ea = jnp.sum(d0 * d0, axis=0, keepdims=True)
        u0 = d0 / (jnp.sqrt(ea) + 1.0)
        for pn in (prm[:_PER_NET], prm[_PER_NET:]):
            emb_w = pn[0][...]
            emb_b = pn[1][...]
            nets.append({"p": pn, "h": emb_w[:, 0:1] * hi + emb_b,
                         "x": x0, "ea": ea, "u0": u0})

    for l in range(NL):
        for st in nets:
            p = st["p"]
            o = 2 + l * _PER_LAYER
            e0w = p[o + 0][...]
            e0b = p[o + 1][...]
            e1w = p[o + 2][...]
            e1b = p[o + 3][...]
            attw = p[o + 4][...]
            attb = p[o + 5][...]
            c0w = p[o + 6][...]
            c0b = p[o + 7][...]
            c1w = p[o + 8][...]
            h = st["h"]
            xc = st["x"]
            ea = st["ea"]

            if l == 0:
                radial, u = ea, st["u0"]
            else:
                d = _dot(xc, s_d)
                radial = jnp.sum(d * d, axis=0, keepdims=True)
                u = d / (jnp.sqrt(radial) + 1.0)

            acat = jnp.concatenate(
                [_dot(e0w[:, :NH], h), _dot(e0w[:, NH:2 * NH], h), e0b],
                axis=1)
            re = jnp.concatenate([radial, ea], axis=0)
            pre = (_dot(acat, s_bc)
                   + _dot(e0w[:, 2 * NH:], re))
            e1 = jax.nn.silu(pre)
            e2 = jax.nn.silu(_dot(e1w, e1) + e1b)
            att = jax.nn.sigmoid(_dot(attw, e2) + attb)
            ef = e2 * att
            m1 = jax.nn.silu(_dot(c0w, ef) + c0b)
            m2 = jnp.tanh(_dot(c1w, m1))

            if l < NL - 1:
                y = _dot(jnp.concatenate([ef, u * m2], axis=0), s_agg_t)
                st["x"] = xc + y[NH:NH + ND] * CRL
                agg_h = y[:NH]
                n0w = p[o + 9][...]
                n0b = p[o + 10][...]
                n1w = p[o + 11][...]
                n1b = p[o + 12][...]
                nh = jax.nn.silu(_dot(n0w[:, :NH], h)
                                 + _dot(n0w[:, NH:], agg_h) + n0b)
                st["h"] = h + _dot(n1w, nh) + n1b
            else:
                st["x"] = xc + _dot(u * m2, s_agg_t) * CRL

    for g in range(GS):
        diff = nets[2 * g]["x"] - nets[2 * g + 1]["x"]
        out_ref[g] = jnp.sum(diff * diff)[None, None]


def kernel(x, h_initial, target_params, predictor_params):
    x3 = x.reshape(NB, NP, ND).transpose(0, 2, 1)
    hi = h_initial.reshape(1, NP)
    s_bc, s_d, s_agg_t = _selection_mats()
    prm = _flatten_net(target_params) + _flatten_net(predictor_params)

    const = lambda shape: pl.BlockSpec(shape, lambda b: (0,) * len(shape))
    in_specs = [
        pl.BlockSpec((GS, ND, NP), lambda b: (b, 0, 0)),
        const((1, NP)),
        const((2 * NP + 1, NPAIR)),
        const((NP, NPAIR)),
        const((NPAIR, NP)),
    ] + [const(p.shape) for p in prm]

    out = pl.pallas_call(
        _body,
        grid=(NB // GS,),
        in_specs=in_specs,
        out_specs=pl.BlockSpec((GS, 1, 1), lambda b: (b, 0, 0)),
        out_shape=jax.ShapeDtypeStruct((NB, 1, 1), jnp.float32),
        compiler_params=pltpu.CompilerParams(
            dimension_semantics=("parallel",),
        ),
    )(x3, hi, s_bc, s_d, s_agg_t, *prm)
    return out[:, 0, 0]
